# baseline (speedup 1.0000x reference)
import jax
import jax.numpy as jnp
from jax.experimental import pallas as pl
from jax.experimental.pallas import tpu as pltpu

_K = 3
_L = 2
_H = 64
_H2 = 72
_N = 2048
_T = _L + _K - 1
_F32 = jnp.float32


def _dot(a, b):
    return jax.lax.dot_general(a, b, (((1,), (0,)), ((), ())),
                               preferred_element_type=_F32)


def _dot_t(a, b):
    return jax.lax.dot_general(a, b, (((0,), (0,)), ((), ())),
                               preferred_element_type=_F32)


def _body_a(xT_ref, a_ref, wcat_ref, bl_ref,
            oA01, oA2, oB01, oB2,
            d1, d2):
    i = pl.program_id(0)
    a = a_ref[0]
    xT = xT_ref[0]

    @pl.when(i > 0)
    def _merged():
        s = jnp.sum(a, axis=1, keepdims=True)
        inv_s = 1.0 / s
        dcat = jnp.concatenate([d1[...], d2[...]], axis=1) * inv_s
        m = _dot(a, dcat)

        @pl.when(i == _K - 1)
        def _emit_a():
            oA01[...] = m

        @pl.when(i == _T - 1)
        def _emit_b():
            oB01[...] = m

        d1[...] = m[:, _H:]

    @pl.when(i == 0)
    def _init():
        d1[...] = jnp.zeros((_N, _H), _F32)

    aggT = _dot(xT, a)
    inv_deg = 1.0 / jnp.maximum(aggT[6:7, :], 1.0)
    cat = jnp.concatenate([aggT * inv_deg, xT], axis=0)
    outT = _dot(wcat_ref[...], cat) + bl_ref[...]
    nsq = jnp.sum(outT * outT, axis=0, keepdims=True)
    inv_n = 1.0 / jnp.maximum(jnp.sqrt(nsq), 1e-12)
    xnT = jnp.maximum(outT * inv_n, 0.0)
    d2[...] = xnT.T

    @pl.when(i == _K - 1)
    def _emit_a2():
        oA2[...] = d2[...]

    @pl.when(i == _T - 1)
    def _emit_b2():
        oB2[...] = d2[...]


def _lstm1_gates(xt, h, w1x_ref, w1h_ref, b1_ref, first):
    g = _dot(xt, w1x_ref[...]) + b1_ref[...]
    if not first:
        g = g + _dot(h, w1h_ref[...])
    return g


def _body_b(dA01, dA2, dB01, dB2, ou2_ref, ou3_ref,
            w1x_ref, w1h_ref, b1_ref, hc1_ref, off1_ref,
            w2a_ref, w2o_ref, w2h_ref, b2_ref, hc2_ref, off2_ref,
            linw_ref, linb_ref, lin1w_ref, lin1b_ref,
            out_ref):
    h2 = jnp.zeros((_N, _H2), _F32)
    c2 = jnp.zeros((_N, _H2), _F32)
    for t, (d01_ref, d2_ref, ou_ref) in enumerate(((dA01, dA2, ou2_ref),
                                                   (dB01, dB2, ou3_ref))):
        d01 = d01_ref[...]
        h = jnp.zeros((_N, _H), _F32)
        c = jnp.zeros((_N, _H), _F32)
        for k, xt in enumerate((d01[:, :_H], d01[:, _H:], d2_ref[...])):
            gates = _lstm1_gates(xt, h, w1x_ref, w1h_ref, b1_ref, k == 0)
            act = jnp.tanh(gates) * hc1_ref[...] + off1_ref[...]
            s_ig = act[:, :2 * _H]
            s_fo = act[:, 2 * _H:]
            c = s_fo[:, :_H] * c + s_ig[:, :_H] * s_ig[:, _H:]
            h = s_fo[:, _H:] * jnp.tanh(c)
        gates2 = _dot(h, w2a_ref[...]) + _dot_t(ou_ref[...], w2o_ref[...]) + b2_ref[...]
        if t > 0:
            gates2 = gates2 + _dot(h2, w2h_ref[...])
        act2 = jnp.tanh(gates2) * hc2_ref[...] + off2_ref[...]
        i2 = act2[:, :_H2]
        g2 = act2[:, 128:128 + _H2]
        f2 = act2[:, 256:256 + _H2]
        o2 = act2[:, 384:384 + _H2]
        c2 = f2 * c2 + i2 * g2
        h2 = o2 * jnp.tanh(c2)
    xl = jnp.maximum(_dot(h2, linw_ref[...]) + linb_ref[...], 0.0)
    out_ref[...] = _dot(xl, lin1w_ref[...]) + lin1b_ref[...]


def kernel(self_loop, x_queue, a_queue, obs_queue, obs_a_queue, u_gamma_queue,
           sage_lin_l_w, sage_lin_l_b, sage_lin_r_w,
           lstm1_w_ih, lstm1_w_hh, lstm1_b_ih, lstm1_b_hh,
           lstm2_w_ih, lstm2_w_hh, lstm2_b_ih, lstm2_b_hh,
           lin_w, lin_b, lin1_w, lin1_b):
    del self_loop, obs_a_queue

    xT_q = jnp.concatenate([
        x_queue.transpose(0, 2, 1),
        jnp.ones((_T, 1, _N), _F32),
        jnp.zeros((_T, 1, _N), _F32),
    ], axis=1)
    bl = sage_lin_l_b.reshape(_H, 1)
    wcat = jnp.concatenate([sage_lin_l_w, jnp.zeros((_H, 2), _F32),
                            sage_lin_r_w, jnp.zeros((_H, 2), _F32)], axis=1)

    full = lambda shape: pl.BlockSpec(shape, lambda i: (0,) * len(shape))
    dA01, dA2, dB01, dB2 = pl.pallas_call(
        _body_a,
        grid=(_T,),
        in_specs=[
            pl.BlockSpec((1, 8, _N), lambda i: (i, 0, 0)),
            pl.BlockSpec((1, _N, _N), lambda i: (i, 0, 0)),
            full((_H, 16)), full((_H, 1)),
        ],
        out_specs=[full((_N, 2 * _H)), full((_N, _H)),
                   full((_N, 2 * _H)), full((_N, _H))],
        out_shape=[jax.ShapeDtypeStruct((_N, 2 * _H), _F32),
                   jax.ShapeDtypeStruct((_N, _H), _F32),
                   jax.ShapeDtypeStruct((_N, 2 * _H), _F32),
                   jax.ShapeDtypeStruct((_N, _H), _F32)],
        scratch_shapes=[pltpu.VMEM((_N, _H), _F32)] * 2,
        compiler_params=pltpu.CompilerParams(
            dimension_semantics=("arbitrary",),
        ),
    )(xT_q, a_queue, wcat, bl)

    ou2 = jnp.concatenate([obs_queue[_K - 1].T, u_gamma_queue[_K - 1].T], axis=0)
    ou3 = jnp.concatenate([obs_queue[_T - 1].T, u_gamma_queue[_T - 1].T], axis=0)

    def regate1(w):
        return jnp.concatenate([0.5 * w[:, :_H], w[:, 2 * _H:3 * _H],
                                0.5 * w[:, _H:2 * _H], 0.5 * w[:, 3 * _H:]],
                               axis=1)

    w1x = regate1(lstm1_w_ih.T)
    w1h = regate1(lstm1_w_hh.T)
    b1 = regate1((lstm1_b_ih + lstm1_b_hh).reshape(1, 4 * _H))
    blk = lambda v: jnp.concatenate(
        [jnp.full((1, _H), v[0], _F32), jnp.full((1, _H), v[1], _F32),
         jnp.full((1, _H), v[2], _F32), jnp.full((1, _H), v[3], _F32)], axis=1)
    hc1 = blk((0.5, 1.0, 0.5, 0.5))
    off1 = blk((0.5, 0.0, 0.5, 0.5))

    def regate2(w):
        z = jnp.zeros((w.shape[0], 512), _F32)
        z = z.at[:, 0:_H2].set(0.5 * w[:, :_H2])
        z = z.at[:, 128:128 + _H2].set(w[:, 2 * _H2:3 * _H2])
        z = z.at[:, 256:256 + _H2].set(0.5 * w[:, _H2:2 * _H2])
        z = z.at[:, 384:384 + _H2].set(0.5 * w[:, 3 * _H2:])
        return z

    w2a = regate2(lstm2_w_ih.T[:_H, :])
    w2o = regate2(lstm2_w_ih.T[_H:, :])
    w2h = regate2(lstm2_w_hh.T)
    b2 = regate2((lstm2_b_ih + lstm2_b_hh).reshape(1, 4 * _H2))
    blk2 = lambda v: jnp.concatenate(
        [jnp.full((1, 128), v[0], _F32), jnp.full((1, 128), v[1], _F32),
         jnp.full((1, 128), v[2], _F32), jnp.full((1, 128), v[3], _F32)], axis=1)
    hc2 = blk2((0.5, 1.0, 0.5, 0.5))
    off2 = blk2((0.5, 0.0, 0.5, 0.5))
    linwT = lin_w.T
    linb = lin_b.reshape(1, _H2)
    lin1wT = lin1_w.T
    lin1b = lin1_b.reshape(1, 2)

    return pl.pallas_call(
        _body_b,
        out_shape=jax.ShapeDtypeStruct((_N, 2), _F32),
    )(dA01, dA2, dB01, dB2, ou2, ou3,
      w1x, w1h, b1, hc1, off1,
      w2a, w2o, w2h, b2, hc2, off2,
      linwT, linb, lin1wT, lin1b)


# speedup vs baseline: 1.2949x; 1.2949x over previous
import jax
import jax.numpy as jnp
from jax.experimental import pallas as pl
from jax.experimental.pallas import tpu as pltpu

_K = 3
_L = 2
_H = 64
_H2 = 72
_N = 2048
_T = _L + _K - 1
_F32 = jnp.float32


def _dot(a, b):
    return jax.lax.dot_general(a, b, (((1,), (0,)), ((), ())),
                               preferred_element_type=_F32)


def _dot_t(a, b):
    return jax.lax.dot_general(a, b, (((0,), (0,)), ((), ())),
                               preferred_element_type=_F32)


def _body_a(xT_ref, a_ref, wcat_ref, bl_ref,
            oA01, oA2, oB01, oB2,
            d1, d2):
    i = pl.program_id(0)
    a = a_ref[0]
    xT = xT_ref[0]

    @pl.when(i > 0)
    def _merged():
        s = jnp.sum(a, axis=1, keepdims=True)
        inv_s = 1.0 / s
        dcat = jnp.concatenate([d1[...], d2[...]], axis=1) * inv_s
        m = _dot(a, dcat)

        @pl.when(i == _K - 1)
        def _emit_a():
            oA01[...] = m

        @pl.when(i == _T - 1)
        def _emit_b():
            oB01[...] = m

        d1[...] = m[:, _H:]

    @pl.when(i == 0)
    def _init():
        d1[...] = jnp.zeros((_N, _H), _F32)

    aggT = _dot(xT, a)
    inv_deg = 1.0 / jnp.maximum(aggT[6:7, :], 1.0)
    cat = jnp.concatenate([aggT * inv_deg, xT], axis=0)
    outT = _dot(wcat_ref[...], cat) + bl_ref[...]
    nsq = jnp.sum(outT * outT, axis=0, keepdims=True)
    inv_n = 1.0 / jnp.maximum(jnp.sqrt(nsq), 1e-12)
    xnT = jnp.maximum(outT * inv_n, 0.0)
    d2[...] = xnT.T

    @pl.when(i == _K - 1)
    def _emit_a2():
        oA2[...] = d2[...]

    @pl.when(i == _T - 1)
    def _emit_b2():
        oB2[...] = d2[...]


def _lstm1_gates(xt, h, w1x_ref, w1h_ref, b1_ref, first):
    g = _dot(xt, w1x_ref[...]) + b1_ref[...]
    if not first:
        g = g + _dot(h, w1h_ref[...])
    return g


def _body_b(dA01, dA2, dB01, dB2, ou2_ref, ou3_ref,
            w1x_ref, w1h_ref, b1_ref,
            w2a_ref, w2o_ref, w2h_ref, b2_ref,
            linw_ref, linb_ref, lin1w_ref, lin1b_ref,
            out_ref):
    hc1 = jnp.concatenate([jnp.full((1, _H), 0.5, _F32),
                           jnp.full((1, _H), 0.5, _F32),
                           jnp.full((1, _H), 1.0, _F32),
                           jnp.full((1, _H), 0.5, _F32)], axis=1)
    off1 = jnp.concatenate([jnp.full((1, _H), 0.5, _F32),
                            jnp.full((1, _H), 0.5, _F32),
                            jnp.full((1, _H), 0.0, _F32),
                            jnp.full((1, _H), 0.5, _F32)], axis=1)
    hc2 = jnp.concatenate([jnp.full((1, _H2), 0.5, _F32),
                           jnp.full((1, _H2), 0.5, _F32),
                           jnp.full((1, _H2), 1.0, _F32),
                           jnp.full((1, _H2), 0.5, _F32)], axis=1)
    off2 = jnp.concatenate([jnp.full((1, _H2), 0.5, _F32),
                            jnp.full((1, _H2), 0.5, _F32),
                            jnp.full((1, _H2), 0.0, _F32),
                            jnp.full((1, _H2), 0.5, _F32)], axis=1)
    h2 = jnp.zeros((_N, _H2), _F32)
    c2 = jnp.zeros((_N, _H2), _F32)
    for t, (d01_ref, d2_ref, ou_ref) in enumerate(((dA01, dA2, ou2_ref),
                                                   (dB01, dB2, ou3_ref))):
        d01 = d01_ref[...]
        h = jnp.zeros((_N, _H), _F32)
        c = jnp.zeros((_N, _H), _F32)
        for k, xt in enumerate((d01[:, :_H], d01[:, _H:], d2_ref[...])):
            gates = _lstm1_gates(xt, h, w1x_ref, w1h_ref, b1_ref, k == 0)
            act = jnp.tanh(gates) * hc1 + off1
            ig = act[:, :_H]
            fg = act[:, _H:2 * _H]
            gg = act[:, 2 * _H:3 * _H]
            og = act[:, 3 * _H:]
            c = fg * c + ig * gg
            h = og * jnp.tanh(c)
        gates2 = _dot(h, w2a_ref[...]) + _dot_t(ou_ref[...], w2o_ref[...]) + b2_ref[...]
        if t > 0:
            gates2 = gates2 + _dot(h2, w2h_ref[...])
        act2 = jnp.tanh(gates2) * hc2 + off2
        ig = act2[:, :_H2]
        fg = act2[:, _H2:2 * _H2]
        gg = act2[:, 2 * _H2:3 * _H2]
        og = act2[:, 3 * _H2:]
        c2 = fg * c2 + ig * gg
        h2 = og * jnp.tanh(c2)
    xl = jnp.maximum(_dot(h2, linw_ref[...]) + linb_ref[...], 0.0)
    out_ref[...] = _dot(xl, lin1w_ref[...]) + lin1b_ref[...]


def kernel(self_loop, x_queue, a_queue, obs_queue, obs_a_queue, u_gamma_queue,
           sage_lin_l_w, sage_lin_l_b, sage_lin_r_w,
           lstm1_w_ih, lstm1_w_hh, lstm1_b_ih, lstm1_b_hh,
           lstm2_w_ih, lstm2_w_hh, lstm2_b_ih, lstm2_b_hh,
           lin_w, lin_b, lin1_w, lin1_b):
    del self_loop, obs_a_queue

    xT_q = jnp.concatenate([
        x_queue.transpose(0, 2, 1),
        jnp.ones((_T, 1, _N), _F32),
        jnp.zeros((_T, 1, _N), _F32),
    ], axis=1)
    bl = sage_lin_l_b.reshape(_H, 1)
    wcat = jnp.concatenate([sage_lin_l_w, jnp.zeros((_H, 2), _F32),
                            sage_lin_r_w, jnp.zeros((_H, 2), _F32)], axis=1)

    full = lambda shape: pl.BlockSpec(shape, lambda i: (0,) * len(shape))
    dA01, dA2, dB01, dB2 = pl.pallas_call(
        _body_a,
        grid=(_T,),
        in_specs=[
            pl.BlockSpec((1, 8, _N), lambda i: (i, 0, 0)),
            pl.BlockSpec((1, _N, _N), lambda i: (i, 0, 0)),
            full((_H, 16)), full((_H, 1)),
        ],
        out_specs=[full((_N, 2 * _H)), full((_N, _H)),
                   full((_N, 2 * _H)), full((_N, _H))],
        out_shape=[jax.ShapeDtypeStruct((_N, 2 * _H), _F32),
                   jax.ShapeDtypeStruct((_N, _H), _F32),
                   jax.ShapeDtypeStruct((_N, 2 * _H), _F32),
                   jax.ShapeDtypeStruct((_N, _H), _F32)],
        scratch_shapes=[pltpu.VMEM((_N, _H), _F32)] * 2,
        compiler_params=pltpu.CompilerParams(
            dimension_semantics=("arbitrary",),
        ),
    )(xT_q, a_queue, wcat, bl)

    ou2 = jnp.concatenate([obs_queue[_K - 1].T, u_gamma_queue[_K - 1].T], axis=0)
    ou3 = jnp.concatenate([obs_queue[_T - 1].T, u_gamma_queue[_T - 1].T], axis=0)

    sc1 = jnp.concatenate([jnp.full((1, _H), 0.5, _F32),
                           jnp.full((1, _H), 0.5, _F32),
                           jnp.full((1, _H), 1.0, _F32),
                           jnp.full((1, _H), 0.5, _F32)], axis=1)
    sc2 = jnp.concatenate([jnp.full((1, _H2), 0.5, _F32),
                           jnp.full((1, _H2), 0.5, _F32),
                           jnp.full((1, _H2), 1.0, _F32),
                           jnp.full((1, _H2), 0.5, _F32)], axis=1)
    w1x = lstm1_w_ih.T * sc1
    w1h = lstm1_w_hh.T * sc1
    b1 = (lstm1_b_ih + lstm1_b_hh).reshape(1, 4 * _H) * sc1
    w2a = lstm2_w_ih.T[:_H, :] * sc2
    w2o = lstm2_w_ih.T[_H:, :] * sc2
    w2h = lstm2_w_hh.T * sc2
    b2 = (lstm2_b_ih + lstm2_b_hh).reshape(1, 4 * _H2) * sc2
    linwT = lin_w.T
    linb = lin_b.reshape(1, _H2)
    lin1wT = lin1_w.T
    lin1b = lin1_b.reshape(1, 2)

    return pl.pallas_call(
        _body_b,
        out_shape=jax.ShapeDtypeStruct((_N, 2), _F32),
    )(dA01, dA2, dB01, dB2, ou2, ou3,
      w1x, w1h, b1,
      w2a, w2o, w2h, b2,
      linwT, linb, lin1wT, lin1b)


# speedup vs baseline: 1.3278x; 1.0254x over previous
import jax
import jax.numpy as jnp
from jax.experimental import pallas as pl
from jax.experimental.pallas import tpu as pltpu

_K = 3
_L = 2
_H = 64
_H2 = 72
_N = 2048
_T = _L + _K - 1
_F32 = jnp.float32


def _dot(a, b):
    return jax.lax.dot_general(a, b, (((1,), (0,)), ((), ())),
                               preferred_element_type=_F32)


def _dot_t(a, b):
    return jax.lax.dot_general(a, b, (((0,), (0,)), ((), ())),
                               preferred_element_type=_F32)


def _body_a(xT_ref, a_ref, wcat_ref, bl_ref,
            oA01, oA2, oB01, oB2,
            d1, d2):
    i = pl.program_id(0)
    a = a_ref[...]
    xT = xT_ref[...]

    @pl.when(i > 0)
    def _merged():
        s = jnp.sum(a, axis=1, keepdims=True)
        inv_s = 1.0 / s
        dcat = jnp.concatenate([d1[...], d2[...]], axis=1) * inv_s
        m = _dot(a, dcat)

        @pl.when(i == _K - 1)
        def _emit_a():
            oA01[...] = m

        @pl.when(i == _T - 1)
        def _emit_b():
            oB01[...] = m

        d1[...] = m[:, _H:]

    @pl.when(i == 0)
    def _init():
        d1[...] = jnp.zeros((_N, _H), _F32)

    aggT = _dot(xT, a)
    inv_deg = 1.0 / jnp.maximum(aggT[6:7, :], 1.0)
    cat = jnp.concatenate([aggT * inv_deg, xT], axis=0)
    outT = _dot(wcat_ref[...], cat) + bl_ref[...]
    nsq = jnp.sum(outT * outT, axis=0, keepdims=True)
    inv_n = 1.0 / jnp.maximum(jnp.sqrt(nsq), 1e-12)
    xnT = jnp.maximum(outT * inv_n, 0.0)
    d2[...] = xnT.T

    @pl.when(i == _K - 1)
    def _emit_a2():
        oA2[...] = d2[...]

    @pl.when(i == _T - 1)
    def _emit_b2():
        oB2[...] = d2[...]


def _lstm1_gates(xt, h, w1x_ref, w1h_ref, b1_ref, first):
    g = _dot(xt, w1x_ref[...]) + b1_ref[...]
    if not first:
        g = g + _dot(h, w1h_ref[...])
    return g


def _body_b(dA01, dA2, dB01, dB2, ou2_ref, ou3_ref,
            w1x_ref, w1h_ref, b1_ref,
            w2a_ref, w2o_ref, w2h_ref, b2_ref,
            linw_ref, linb_ref, lin1w_ref, lin1b_ref,
            out_ref):
    h2 = jnp.zeros((_N, _H2), _F32)
    c2 = jnp.zeros((_N, _H2), _F32)
    for t, (d01_ref, d2_ref, ou_ref) in enumerate(((dA01, dA2, ou2_ref),
                                                   (dB01, dB2, ou3_ref))):
        d01 = d01_ref[...]
        h = jnp.zeros((_N, _H), _F32)
        c = jnp.zeros((_N, _H), _F32)
        for k, xt in enumerate((d01[:, :_H], d01[:, _H:], d2_ref[...])):
            gates = _lstm1_gates(xt, h, w1x_ref, w1h_ref, b1_ref, k == 0)
            ig = jax.nn.sigmoid(gates[:, :_H])
            fg = jax.nn.sigmoid(gates[:, _H:2 * _H])
            gg = jnp.tanh(gates[:, 2 * _H:3 * _H])
            og = jax.nn.sigmoid(gates[:, 3 * _H:])
            c = fg * c + ig * gg
            h = og * jnp.tanh(c)
        gates2 = _dot(h, w2a_ref[...]) + _dot_t(ou_ref[...], w2o_ref[...]) + b2_ref[...]
        if t > 0:
            gates2 = gates2 + _dot(h2, w2h_ref[...])
        ig = jax.nn.sigmoid(gates2[:, :_H2])
        fg = jax.nn.sigmoid(gates2[:, _H2:2 * _H2])
        gg = jnp.tanh(gates2[:, 2 * _H2:3 * _H2])
        og = jax.nn.sigmoid(gates2[:, 3 * _H2:])
        c2 = fg * c2 + ig * gg
        h2 = og * jnp.tanh(c2)
    xl = jnp.maximum(_dot(h2, linw_ref[...]) + linb_ref[...], 0.0)
    out_ref[...] = _dot(xl, lin1w_ref[...]) + lin1b_ref[...]


def kernel(self_loop, x_queue, a_queue, obs_queue, obs_a_queue, u_gamma_queue,
           sage_lin_l_w, sage_lin_l_b, sage_lin_r_w,
           lstm1_w_ih, lstm1_w_hh, lstm1_b_ih, lstm1_b_hh,
           lstm2_w_ih, lstm2_w_hh, lstm2_b_ih, lstm2_b_hh,
           lin_w, lin_b, lin1_w, lin1_b):
    del self_loop, obs_a_queue

    xT_q = jnp.concatenate([
        x_queue.transpose(0, 2, 1),
        jnp.ones((_T, 1, _N), _F32),
        jnp.zeros((_T, 1, _N), _F32),
    ], axis=1)
    bl = sage_lin_l_b.reshape(_H, 1)
    wcat = jnp.concatenate([sage_lin_l_w, jnp.zeros((_H, 2), _F32),
                            sage_lin_r_w, jnp.zeros((_H, 2), _F32)], axis=1)

    full = lambda shape: pl.BlockSpec(shape, lambda i: (0,) * len(shape))
    dA01, dA2, dB01, dB2 = pl.pallas_call(
        _body_a,
        grid=(_T,),
        in_specs=[
            pl.BlockSpec((None, 8, _N), lambda i: (i, 0, 0)),
            pl.BlockSpec((None, _N, _N), lambda i: (i, 0, 0)),
            full((_H, 16)), full((_H, 1)),
        ],
        out_specs=[full((_N, 2 * _H)), full((_N, _H)),
                   full((_N, 2 * _H)), full((_N, _H))],
        out_shape=[jax.ShapeDtypeStruct((_N, 2 * _H), _F32),
                   jax.ShapeDtypeStruct((_N, _H), _F32),
                   jax.ShapeDtypeStruct((_N, 2 * _H), _F32),
                   jax.ShapeDtypeStruct((_N, _H), _F32)],
        scratch_shapes=[pltpu.VMEM((_N, _H), _F32)] * 2,
        compiler_params=pltpu.CompilerParams(
            dimension_semantics=("arbitrary",),
        ),
    )(xT_q, a_queue, wcat, bl)

    ou2 = jnp.concatenate([obs_queue[_K - 1].T, u_gamma_queue[_K - 1].T], axis=0)
    ou3 = jnp.concatenate([obs_queue[_T - 1].T, u_gamma_queue[_T - 1].T], axis=0)

    w1x = lstm1_w_ih.T
    w1h = lstm1_w_hh.T
    b1 = (lstm1_b_ih + lstm1_b_hh).reshape(1, 4 * _H)
    w2a = lstm2_w_ih.T[:_H, :]
    w2o = lstm2_w_ih.T[_H:, :]
    w2h = lstm2_w_hh.T
    b2 = (lstm2_b_ih + lstm2_b_hh).reshape(1, 4 * _H2)
    linwT = lin_w.T
    linb = lin_b.reshape(1, _H2)
    lin1wT = lin1_w.T
    lin1b = lin1_b.reshape(1, 2)

    return pl.pallas_call(
        _body_b,
        out_shape=jax.ShapeDtypeStruct((_N, 2), _F32),
    )(dA01, dA2, dB01, dB2, ou2, ou3,
      w1x, w1h, b1,
      w2a, w2o, w2h, b2,
      linwT, linb, lin1wT, lin1b)
